# baseline (speedup 1.0000x reference)
import functools

import jax
import jax.numpy as jnp
from jax import lax
from jax.experimental import pallas as pl
from jax.experimental.pallas import tpu as pltpu
from jax.experimental.pallas import tpu_sc as plsc

_TRUNC = 32
_NEG_SENTINEL = -1.0

_NC = 2
_NS = 16
_LANES = 16
_NW = _NC * _NS


def _topk_kernel(x_ref, vals_ref, idx_ref):
    x = x_ref[...]
    b, n = x.shape
    col = lax.broadcasted_iota(jnp.int32, (b, n), 1)
    kcol = lax.broadcasted_iota(jnp.int32, (b, _TRUNC), 1)

    def body(k, carry):
        xm, vacc, iacc = carry
        rowmax = jnp.max(xm, axis=1, keepdims=True)
        logv = jnp.clip(jnp.log(rowmax), -1000.0, None) + 50.0
        poscand = jnp.where(xm == rowmax, col, n)
        firstpos = jnp.min(poscand, axis=1, keepdims=True)
        mask = col == firstpos
        ksel = kcol == k
        vacc = jnp.where(ksel, logv, vacc)
        iacc = jnp.where(ksel, firstpos, iacc)
        xm = jnp.where(mask, _NEG_SENTINEL, xm)
        return xm, vacc, iacc

    vacc0 = jnp.zeros((b, _TRUNC), jnp.float32)
    iacc0 = jnp.zeros((b, _TRUNC), jnp.int32)
    _, vacc, iacc = lax.fori_loop(0, _TRUNC, body, (x, vacc0, iacc0))
    shift = jax.nn.relu(-jnp.min(vacc, axis=1, keepdims=True))
    v = vacc + shift
    norm = jnp.sqrt(jnp.sum(v * v, axis=1, keepdims=True))
    vals_ref[...] = v / jnp.clip(norm, 1e-12, None)
    idx_ref[...] = iacc


def _sc_body(idx_hbm, val_hbm, b1_hbm, w1_hbm, out_hbm,
             idx_v, val_v, b1_v, h_v, idxbuf, gbuf, sem):
    cid = lax.axis_index("c")
    sid = lax.axis_index("s")
    wid = sid * _NC + cid
    i_base = wid * 256

    pltpu.sync_copy(idx_hbm, idx_v)
    pltpu.sync_copy(val_hbm, val_v)
    pltpu.sync_copy(b1_hbm.at[pl.ds(i_base, 256)], b1_v)

    lane = lax.iota(jnp.int32, _LANES)

    for half in range(2):
        ibase_h = i_base + half * 128
        rts = [(ibase_h + ci * 16 + lane) * 8192 for ci in range(8)]

        for bg in range(2):
            bk0 = bg * 128

            def genbody(bk, carry, rts=rts, bk0=bk0):
                jv = idx_v[bk0 + bk, pl.ds(0, 16)]
                for ci in range(8):
                    idxbuf[bk, pl.ds(ci * 16, 16)] = rts[ci] + jv
                return carry

            lax.fori_loop(0, 128, genbody, 0)

            def fire(bk, carry):
                pltpu.async_copy(w1_hbm.at[idxbuf.at[bk]], gbuf.at[bk], sem)
                return carry

            lax.fori_loop(0, 128, fire, 0)

            def drain(bk, carry):
                pltpu.make_async_copy(
                    w1_hbm.at[idxbuf.at[bk]], gbuf.at[bk], sem).wait()
                return carry

            lax.fori_loop(0, 128, drain, 0)

            for bl in range(4):
                b = bg * 4 + bl

                def fmabody(k, acc, bl=bl, bk0=bk0):
                    bk = bl * _TRUNC + k
                    vv = val_v[bk0 + bk, pl.ds(0, 16)]
                    return tuple(acc[ci] + vv * gbuf[bk, pl.ds(ci * 16, 16)]
                                 for ci in range(8))

                acc0 = tuple(jnp.zeros((_LANES,), jnp.float32) for _ in range(8))
                acc = lax.fori_loop(0, _TRUNC, fmabody, acc0)
                for ci in range(8):
                    off = half * 128 + ci * 16
                    hb = acc[ci] + b1_v[pl.ds(off, 16)]
                    h_v[b, pl.ds(off, 16)] = jnp.where(hb >= 0.0, hb, 0.2 * hb)

    pltpu.sync_copy(h_v, out_hbm.at[pl.ds(0, 8), pl.ds(i_base, 256)])


@functools.cache
def _sc_gather_mlp():
    return pl.kernel(
        _sc_body,
        out_type=jax.ShapeDtypeStruct((8, 8192), jnp.float32),
        mesh=plsc.VectorSubcoreMesh(
            core_axis_name="c", subcore_axis_name="s",
            num_cores=_NC, num_subcores=_NS),
        scratch_types=[
            pltpu.VMEM((256, 16), jnp.int32),
            pltpu.VMEM((256, 16), jnp.float32),
            pltpu.VMEM((256,), jnp.float32),
            pltpu.VMEM((8, 256), jnp.float32),
            pltpu.VMEM((128, 128), jnp.int32),
            pltpu.VMEM((128, 128), jnp.float32),
            pltpu.SemaphoreType.DMA,
        ],
    )


def _out_kernel(h_ref, w2_ref, b2_ref, out_ref, acc_ref):
    j = pl.program_id(0)

    @pl.when(j == 0)
    def _():
        acc_ref[...] = jnp.zeros_like(acc_ref)

    acc_ref[...] += lax.dot_general(
        h_ref[...], w2_ref[...], (((1,), (1,)), ((), ())),
        preferred_element_type=jnp.float32)

    @pl.when(j == pl.num_programs(0) - 1)
    def _():
        out_ref[...] = acc_ref[...] + b2_ref[...]


@jax.jit
def _impl(x, W1, b1, W2, b2):
    batch, n = x.shape
    out_dim = W2.shape[0]

    vals, idx = pl.pallas_call(
        _topk_kernel,
        out_shape=(
            jax.ShapeDtypeStruct((batch, _TRUNC), jnp.float32),
            jax.ShapeDtypeStruct((batch, _TRUNC), jnp.int32),
        ),
    )(x)

    nbk = batch * _TRUNC
    idx_b = jnp.broadcast_to(idx.reshape(nbk, 1), (nbk, _LANES))
    val_b = jnp.broadcast_to(vals.reshape(nbk, 1), (nbk, _LANES))
    h_act = _sc_gather_mlp()(idx_b, val_b, b1, W1.reshape(-1))

    blk = 512
    out = pl.pallas_call(
        _out_kernel,
        grid=(n // blk,),
        in_specs=[
            pl.BlockSpec((batch, blk), lambda j: (0, j)),
            pl.BlockSpec((out_dim, blk), lambda j: (0, j)),
            pl.BlockSpec((1, out_dim), lambda j: (0, 0)),
        ],
        out_specs=pl.BlockSpec((batch, out_dim), lambda j: (0, 0)),
        out_shape=jax.ShapeDtypeStruct((batch, out_dim), jnp.float32),
        scratch_shapes=[pltpu.VMEM((batch, out_dim), jnp.float32)],
    )(h_act, W2, b2.reshape(1, -1))
    return out


def kernel(x, W1, b1, W2, b2):
    return _impl(x, W1, b1, W2, b2)


# speedup vs baseline: 2.3001x; 2.3001x over previous
import functools

import jax
import jax.numpy as jnp
from jax import lax
from jax.experimental import pallas as pl
from jax.experimental.pallas import tpu as pltpu
from jax.experimental.pallas import tpu_sc as plsc

_TRUNC = 32
_NEG_SENTINEL = -1.0

_NC = 2
_NS = 16
_LANES = 16
_NW = _NC * _NS

_N = 8192
_R = 4608
_ROWS_PT = (_N - _R) // _NW
_NBANDS = _ROWS_PT // 8
_BLK = 512


def _topk_kernel(x_ref, vals_ref, idx_ref, z_ref):
    x = x_ref[...]
    b, n = x.shape
    col = lax.broadcasted_iota(jnp.int32, (b, n), 1)
    kcol = lax.broadcasted_iota(jnp.int32, (b, _TRUNC), 1)

    def body(k, carry):
        xm, zlog, sel, vacc, iacc = carry
        rowmax = jnp.max(xm, axis=1, keepdims=True)
        logv = jnp.clip(jnp.log(rowmax), -1000.0, None) + 50.0
        poscand = jnp.where(xm == rowmax, col, n)
        firstpos = jnp.min(poscand, axis=1, keepdims=True)
        mask = col == firstpos
        ksel = kcol == k
        vacc = jnp.where(ksel, logv, vacc)
        iacc = jnp.where(ksel, firstpos, iacc)
        zlog = zlog + jnp.where(mask, logv, 0.0)
        sel = sel + jnp.where(mask, 1.0, 0.0)
        xm = jnp.where(mask, _NEG_SENTINEL, xm)
        return xm, zlog, sel, vacc, iacc

    zeros = jnp.zeros((b, n), jnp.float32)
    vacc0 = jnp.zeros((b, _TRUNC), jnp.float32)
    iacc0 = jnp.zeros((b, _TRUNC), jnp.int32)
    _, zlog, sel, vacc, iacc = lax.fori_loop(
        0, _TRUNC, body, (x, zeros, zeros, vacc0, iacc0))
    shift = jax.nn.relu(-jnp.min(vacc, axis=1, keepdims=True))
    v = vacc + shift
    norm = jnp.clip(jnp.sqrt(jnp.sum(v * v, axis=1, keepdims=True)),
                    1e-12, None)
    vals_ref[...] = v / norm
    idx_ref[...] = iacc
    z_ref[...] = sel * (zlog + shift) / norm


def _sc_body(idx_hbm, val_hbm, w1_hbm, out_hbm, idx_v, val_v, band, hacc, sem):
    cid = lax.axis_index("c")
    sid = lax.axis_index("s")
    wid = sid * _NC + cid
    row0 = _R + wid * _ROWS_PT

    pltpu.sync_copy(idx_hbm, idx_v)
    pltpu.sync_copy(val_hbm, val_v)
    lane = lax.iota(jnp.int32, _LANES)

    def bandloop(bi, carry):
        src = w1_hbm.at[pl.ds(row0 + bi * 8, 8), pl.ds(0, _N)]
        pltpu.async_copy(src, band, sem)
        pltpu.make_async_copy(src, band, sem).wait()

        def rowloop(r, carry2):
            iv = jnp.full((_LANES,), r, jnp.int32)
            sums = []
            for b in range(8):
                parts = None
                for c in range(2):
                    bk0 = b * _TRUNC + c * 16
                    jv = idx_v[pl.ds(bk0, 16)]
                    vv = val_v[pl.ds(bk0, 16)]
                    g = plsc.load_gather(band, [iv, jv])
                    gv = g * vv
                    parts = gv if parts is None else parts + gv
                s = plsc.cumsum(parts)
                sums.append(s[15])
            hv = jnp.zeros((_LANES,), jnp.float32)
            for b in range(8):
                hv = jnp.where(lane == b, sums[b], hv)
            plsc.store_scatter(
                hacc, [jnp.full((_LANES,), bi * 8 + r, jnp.int32), lane],
                hv, mask=lane < 8)
            return carry2

        lax.fori_loop(0, 8, rowloop, 0)
        return carry

    lax.fori_loop(0, _NBANDS, bandloop, 0)
    pltpu.sync_copy(hacc, out_hbm.at[pl.ds(wid * _ROWS_PT, _ROWS_PT),
                                     pl.ds(0, 8)])


@functools.cache
def _sc_gather_mlp():
    return pl.kernel(
        _sc_body,
        out_type=jax.ShapeDtypeStruct((_N - _R, 8), jnp.float32),
        mesh=plsc.VectorSubcoreMesh(
            core_axis_name="c", subcore_axis_name="s",
            num_cores=_NC, num_subcores=_NS),
        scratch_types=[
            pltpu.VMEM((_TRUNC * 8,), jnp.int32),
            pltpu.VMEM((_TRUNC * 8,), jnp.float32),
            pltpu.VMEM((8, _N), jnp.float32),
            pltpu.VMEM((_ROWS_PT, 8), jnp.float32),
            pltpu.SemaphoreType.DMA,
        ],
        compiler_params=pltpu.CompilerParams(
            use_tc_tiling_on_sc=True, needs_layout_passes=False),
    )


def _mlp_part1(z_ref, w1_ref, b1_ref, w2_ref, out_ref, acc_ref):
    j = pl.program_id(0)

    @pl.when(j == 0)
    def _():
        acc_ref[...] = jnp.zeros_like(acc_ref)

    h = lax.dot_general(
        z_ref[...], w1_ref[...], (((1,), (1,)), ((), ())),
        preferred_element_type=jnp.float32) + b1_ref[...]
    h = jnp.where(h >= 0, h, 0.2 * h)
    acc_ref[...] += lax.dot_general(
        h, w2_ref[...], (((1,), (1,)), ((), ())),
        preferred_element_type=jnp.float32)

    @pl.when(j == pl.num_programs(0) - 1)
    def _():
        out_ref[...] = acc_ref[...]


def _mlp_part2(ht_ref, b1c_ref, w2_ref, part_ref, b2_ref, out_ref, acc_ref):
    j = pl.program_id(0)

    @pl.when(j == 0)
    def _():
        acc_ref[...] = jnp.zeros_like(acc_ref)

    h = ht_ref[...] + b1c_ref[pl.ds(j * _BLK, _BLK), :]
    h = jnp.where(h >= 0, h, 0.2 * h)
    acc_ref[...] += lax.dot_general(
        h, w2_ref[...], (((0,), (1,)), ((), ())),
        preferred_element_type=jnp.float32)

    @pl.when(j == pl.num_programs(0) - 1)
    def _():
        out_ref[...] = acc_ref[...] + part_ref[...] + b2_ref[...]


@jax.jit
def _impl(x, W1, b1, W2, b2):
    batch, n = x.shape
    out_dim = W2.shape[0]

    vals, idx, z = pl.pallas_call(
        _topk_kernel,
        out_shape=(
            jax.ShapeDtypeStruct((batch, _TRUNC), jnp.float32),
            jax.ShapeDtypeStruct((batch, _TRUNC), jnp.int32),
            jax.ShapeDtypeStruct((batch, n), jnp.float32),
        ),
    )(x)

    h_sc_t = _sc_gather_mlp()(idx.reshape(-1), vals.reshape(-1), W1)

    b1r = b1.reshape(1, -1)
    part = pl.pallas_call(
        _mlp_part1,
        grid=(_R // _BLK,),
        in_specs=[
            pl.BlockSpec((batch, n), lambda j: (0, 0)),
            pl.BlockSpec((_BLK, n), lambda j: (j, 0)),
            pl.BlockSpec((1, _BLK), lambda j: (0, j)),
            pl.BlockSpec((out_dim, _BLK), lambda j: (0, j)),
        ],
        out_specs=pl.BlockSpec((batch, out_dim), lambda j: (0, 0)),
        out_shape=jax.ShapeDtypeStruct((batch, out_dim), jnp.float32),
        scratch_shapes=[pltpu.VMEM((batch, out_dim), jnp.float32)],
    )(z, W1, b1r, W2)

    nblk2 = (n - _R) // _BLK
    b1col = b1[_R:].reshape(-1, 1)
    out = pl.pallas_call(
        _mlp_part2,
        grid=(nblk2,),
        in_specs=[
            pl.BlockSpec((_BLK, batch), lambda j: (j, 0)),
            pl.BlockSpec((n - _R, 1), lambda j: (0, 0)),
            pl.BlockSpec((out_dim, _BLK), lambda j: (0, (_R // _BLK) + j)),
            pl.BlockSpec((batch, out_dim), lambda j: (0, 0)),
            pl.BlockSpec((1, out_dim), lambda j: (0, 0)),
        ],
        out_specs=pl.BlockSpec((batch, out_dim), lambda j: (0, 0)),
        out_shape=jax.ShapeDtypeStruct((batch, out_dim), jnp.float32),
        scratch_shapes=[pltpu.VMEM((batch, out_dim), jnp.float32)],
    )(h_sc_t, b1col, W2, part, b2.reshape(1, -1))
    return out


def kernel(x, W1, b1, W2, b2):
    return _impl(x, W1, b1, W2, b2)
